# baseline (speedup 1.0000x reference)
import functools

import jax
import jax.numpy as jnp
from jax import lax
from jax.experimental import pallas as pl
from jax.experimental.pallas import tpu as pltpu
from jax.experimental.pallas import tpu_sc as plsc

N_NODES = 10000
IN_CH = 128
HID = 32
N_LAYERS = 4
N_CLASSES = 2
N_EDGES = 320000
BN_EPS = 1e-5

_NC = 2
_NS = 16
_NW = _NC * _NS

_CHUNK = 128
_EPT = 10240
_EPAD = _EPT * _NW
_NCH = _EPT // _CHUNK
_NPAD = 10112
_ZR = _NPAD // _NS
_GSZ = 512
_NST = _EPT // _GSZ
_NBUF = 4
_AHEAD = 2


def _sc_agg_body(y_hbm, srcm_hbm, dstm_hbm, zeros_hbm, out_hbm,
                 s_all, d_all, rows, agg_sh, gsem, ssem):
    scid = lax.axis_index("c")
    sid = lax.axis_index("s")
    wid = scid * _NS + sid
    ebase = wid * _EPT

    pltpu.sync_copy(srcm_hbm.at[pl.ds(ebase, _EPT)], s_all)
    pltpu.sync_copy(dstm_hbm.at[pl.ds(ebase, _EPT)], d_all)
    pltpu.sync_copy(zeros_hbm.at[pl.ds(sid * _ZR, _ZR)],
                    agg_sh.at[pl.ds(sid * _ZR, _ZR)])
    plsc.subcore_barrier()

    def g_issue(c, b):
        pltpu.async_copy(y_hbm.at[s_all.at[pl.ds(c * _GSZ, _GSZ)]],
                         rows.at[b], gsem.at[b])

    def g_wait(c, b):
        pltpu.make_async_copy(y_hbm.at[s_all.at[pl.ds(c * _GSZ, _GSZ)]],
                              rows.at[b], gsem.at[b]).wait()

    def s_issue(c, b):
        pltpu.async_copy(rows.at[b],
                         agg_sh.at[d_all.at[pl.ds(c * _GSZ, _GSZ)]],
                         ssem.at[b], add=True)

    def s_wait(c, b):
        pltpu.make_async_copy(rows.at[b],
                              agg_sh.at[d_all.at[pl.ds(c * _GSZ, _GSZ)]],
                              ssem.at[b]).wait()

    for k in range(_AHEAD):
        g_issue(k, k)

    def group(gi, carry):
        c0 = gi * _NBUF
        for b in range(_NBUF):
            c = c0 + b
            g_wait(c, b)
            s_issue(c, b)
            bp = (b + _AHEAD) % _NBUF

            @pl.when(c + _AHEAD < _NST)
            def _():
                @pl.when(c >= _AHEAD)
                def _():
                    s_wait(c - _AHEAD, bp)

                g_issue(c + _AHEAD, bp)

        return carry

    lax.fori_loop(0, _NST // _NBUF, group, 0)
    for b in range(_NBUF):
        s_wait(_NST - _NBUF + b, b)
    plsc.subcore_barrier()

    r0 = sid * _ZR
    pltpu.sync_copy(agg_sh.at[pl.ds(r0, _ZR)],
                    out_hbm.at[scid, pl.ds(r0, _ZR)])


@functools.cache
def _sc_agg_call():
    return pl.kernel(
        _sc_agg_body,
        out_type=jax.ShapeDtypeStruct((_NC, _NPAD, HID), jnp.float32),
        mesh=plsc.VectorSubcoreMesh(core_axis_name="c",
                                    subcore_axis_name="s"),
        compiler_params=pltpu.CompilerParams(use_tc_tiling_on_sc=False),
        scratch_types=[
            pltpu.VMEM((_EPT,), jnp.int32),
            pltpu.VMEM((_EPT,), jnp.int32),
            pltpu.VMEM((_NBUF, _GSZ, HID), jnp.float32),
            pltpu.VMEM_SHARED((_NPAD, HID), jnp.float32),
            pltpu.SemaphoreType.DMA((_NBUF,)),
            pltpu.SemaphoreType.DMA((_NBUF,)),
        ],
    )


def _sc_agg(y, srcm, dstm, zeros_init):
    full = _sc_agg_call()(y, srcm, dstm, zeros_init)
    return full[:, :N_NODES]


def _mm_body(h_ref, w_ref, o_ref):
    o_ref[...] = jnp.dot(h_ref[...], w_ref[...],
                         preferred_element_type=jnp.float32,
                 precision=lax.Precision.HIGHEST)


def _bn_relu(t, g, b):
    mu = jnp.mean(t, axis=0, keepdims=True)
    d = t - mu
    var = jnp.mean(d * d, axis=0, keepdims=True)
    return jnp.maximum(g * d / jnp.sqrt(var + BN_EPS) + b, 0.0)


def _layer_body(y_ref, a0_ref, a1_ref, b1_ref, g1_ref, be1_ref,
                w2_ref, b2_ref, g2_ref, be2_ref, w1n_ref,
                z2_ref, yn_ref):
    t = y_ref[...] + a0_ref[...] + a1_ref[...] + b1_ref[...]
    z1 = _bn_relu(t, g1_ref[...], be1_ref[...])
    u = jnp.dot(z1, w2_ref[...], preferred_element_type=jnp.float32,
                 precision=lax.Precision.HIGHEST) \
        + b2_ref[...]
    z2 = _bn_relu(u, g2_ref[...], be2_ref[...])
    z2_ref[...] = z2
    yn_ref[...] = jnp.dot(z2, w1n_ref[...],
                          preferred_element_type=jnp.float32,
                 precision=lax.Precision.HIGHEST)


def _last_body(y_ref, a0_ref, a1_ref, b1_ref, g1_ref, be1_ref,
               w2_ref, b2_ref, g2_ref, be2_ref,
               x_ref, z0_ref, z1_ref, z2_ref,
               wox_ref, wo0_ref, wo1_ref, wo2_ref, wo3_ref, bo_ref,
               pred_ref):
    t = y_ref[...] + a0_ref[...] + a1_ref[...] + b1_ref[...]
    zz1 = _bn_relu(t, g1_ref[...], be1_ref[...])
    u = jnp.dot(zz1, w2_ref[...], preferred_element_type=jnp.float32) \
        + b2_ref[...]
    z3 = _bn_relu(u, g2_ref[...], be2_ref[...])
    pred = bo_ref[...]
    for lhs, w in ((x_ref[...], wox_ref[...]), (z0_ref[...], wo0_ref[...]),
                   (z1_ref[...], wo1_ref[...]), (z2_ref[...], wo2_ref[...]),
                   (z3, wo3_ref[...])):
        pred = pred + jnp.dot(lhs, w, preferred_element_type=jnp.float32)
    pred_ref[...] = pred


def _tc(body, out_shapes):
    return pl.pallas_call(body, out_shape=out_shapes)


def kernel(x, edge_index, params):
    src = edge_index[0]
    dst = edge_index[1]
    npad = _EPAD - N_EDGES
    src_p = jnp.concatenate([src, jnp.zeros((npad,), jnp.int32)])
    dst_p = jnp.concatenate([dst, jnp.full((npad,), N_NODES, jnp.int32)])
    zeros_init = jnp.zeros((_NPAD, HID), jnp.float32)

    layers = params["layers"]
    nd = lambda a: a.reshape(1, -1)

    y = _tc(_mm_body, jax.ShapeDtypeStruct((N_NODES, HID), jnp.float32))(
        x, layers[0]["W1"])

    zs = []
    for i in range(N_LAYERS - 1):
        lp = layers[i]
        agg2 = _sc_agg(y, src_p, dst_p, zeros_init)
        z2, y = _tc(_layer_body, (
            jax.ShapeDtypeStruct((N_NODES, HID), jnp.float32),
            jax.ShapeDtypeStruct((N_NODES, HID), jnp.float32),
        ))(y, agg2[0], agg2[1], nd(lp["b1"]), nd(lp["bn1_g"]),
           nd(lp["bn1_b"]), lp["W2"], nd(lp["b2"]), nd(lp["bn2_g"]),
           nd(lp["bn2_b"]), layers[i + 1]["W1"])
        zs.append(z2)

    lp = layers[N_LAYERS - 1]
    agg2 = _sc_agg(y, src_p, dst_p, zeros_init)
    w_out = params["W_out"]
    pred = _tc(_last_body, jax.ShapeDtypeStruct(
        (N_NODES, N_CLASSES), jnp.float32))(
        y, agg2[0], agg2[1], nd(lp["b1"]), nd(lp["bn1_g"]), nd(lp["bn1_b"]),
        lp["W2"], nd(lp["b2"]), nd(lp["bn2_g"]), nd(lp["bn2_b"]),
        x, zs[0], zs[1], zs[2],
        w_out[:IN_CH], w_out[IN_CH:IN_CH + HID],
        w_out[IN_CH + HID:IN_CH + 2 * HID],
        w_out[IN_CH + 2 * HID:IN_CH + 3 * HID],
        w_out[IN_CH + 3 * HID:],
        params["b_out"].reshape(1, -1))
    return pred


# speedup vs baseline: 1.8893x; 1.8893x over previous
import functools

import jax
import jax.numpy as jnp
from jax import lax
from jax.experimental import pallas as pl
from jax.experimental.pallas import tpu as pltpu
from jax.experimental.pallas import tpu_sc as plsc

N_NODES = 10000
IN_CH = 128
HID = 32
N_LAYERS = 4
N_CLASSES = 2
N_EDGES = 320000
BN_EPS = 1e-5

_NC = 2
_NS = 16
_NW = _NC * _NS

_CHUNK = 128
_EPT = 10240
_EPAD = _EPT * _NW
_NCH = _EPT // _CHUNK
_NPAD = 10112
_ZR = _NPAD // _NS
_GSZ = 512
_NST = _EPT // _GSZ
_NBUF = 4
_AHEAD = 2


def _sc_agg_body(y_hbm, srcm_hbm, dstm_hbm, zeros_hbm, out_hbm,
                 s_all, d_all, rows, agg_sh, y_sh, gsem, ssem):
    scid = lax.axis_index("c")
    sid = lax.axis_index("s")
    wid = scid * _NS + sid
    ebase = wid * _EPT

    pltpu.sync_copy(srcm_hbm.at[pl.ds(ebase, _EPT)], s_all)
    pltpu.sync_copy(dstm_hbm.at[pl.ds(ebase, _EPT)], d_all)
    pltpu.sync_copy(zeros_hbm.at[pl.ds(sid * _ZR, _ZR)],
                    agg_sh.at[pl.ds(sid * _ZR, _ZR)])
    yr = N_NODES // _NS
    pltpu.sync_copy(y_hbm.at[pl.ds(sid * yr, yr)],
                    y_sh.at[pl.ds(sid * yr, yr)])
    plsc.subcore_barrier()

    def g_issue(c, b):
        pltpu.async_copy(y_sh.at[s_all.at[pl.ds(c * _GSZ, _GSZ)]],
                         rows.at[b], gsem.at[b])

    def g_wait(c, b):
        pltpu.make_async_copy(y_sh.at[s_all.at[pl.ds(c * _GSZ, _GSZ)]],
                              rows.at[b], gsem.at[b]).wait()

    def s_issue(c, b):
        pltpu.async_copy(rows.at[b],
                         agg_sh.at[d_all.at[pl.ds(c * _GSZ, _GSZ)]],
                         ssem.at[b], add=True)

    def s_wait(c, b):
        pltpu.make_async_copy(rows.at[b],
                              agg_sh.at[d_all.at[pl.ds(c * _GSZ, _GSZ)]],
                              ssem.at[b]).wait()

    for k in range(_AHEAD):
        g_issue(k, k)

    def group(gi, carry):
        c0 = gi * _NBUF
        for b in range(_NBUF):
            c = c0 + b
            g_wait(c, b)
            s_issue(c, b)
            bp = (b + _AHEAD) % _NBUF

            @pl.when(c + _AHEAD < _NST)
            def _():
                @pl.when(c >= _AHEAD)
                def _():
                    s_wait(c - _AHEAD, bp)

                g_issue(c + _AHEAD, bp)

        return carry

    lax.fori_loop(0, _NST // _NBUF, group, 0)
    for b in range(_NBUF):
        s_wait(_NST - _NBUF + b, b)
    plsc.subcore_barrier()

    r0 = sid * _ZR
    pltpu.sync_copy(agg_sh.at[pl.ds(r0, _ZR)],
                    out_hbm.at[scid, pl.ds(r0, _ZR)])


@functools.cache
def _sc_agg_call():
    return pl.kernel(
        _sc_agg_body,
        out_type=jax.ShapeDtypeStruct((_NC, _NPAD, HID), jnp.float32),
        mesh=plsc.VectorSubcoreMesh(core_axis_name="c",
                                    subcore_axis_name="s"),
        compiler_params=pltpu.CompilerParams(use_tc_tiling_on_sc=False),
        scratch_types=[
            pltpu.VMEM((_EPT,), jnp.int32),
            pltpu.VMEM((_EPT,), jnp.int32),
            pltpu.VMEM((_NBUF, _GSZ, HID), jnp.float32),
            pltpu.VMEM_SHARED((_NPAD, HID), jnp.float32),
            pltpu.VMEM_SHARED((N_NODES, HID), jnp.float32),
            pltpu.SemaphoreType.DMA((_NBUF,)),
            pltpu.SemaphoreType.DMA((_NBUF,)),
        ],
    )


def _sc_agg(y, srcm, dstm, zeros_init):
    full = _sc_agg_call()(y, srcm, dstm, zeros_init)
    return full[:, :N_NODES]


def _mm_body(h_ref, w_ref, o_ref):
    o_ref[...] = jnp.dot(h_ref[...], w_ref[...],
                         preferred_element_type=jnp.float32,
                 precision=lax.Precision.HIGHEST)


def _bn_relu(t, g, b):
    mu = jnp.mean(t, axis=0, keepdims=True)
    d = t - mu
    var = jnp.mean(d * d, axis=0, keepdims=True)
    return jnp.maximum(g * d / jnp.sqrt(var + BN_EPS) + b, 0.0)


def _layer_body(y_ref, a0_ref, a1_ref, b1_ref, g1_ref, be1_ref,
                w2_ref, b2_ref, g2_ref, be2_ref, w1n_ref,
                z2_ref, yn_ref):
    t = y_ref[...] + a0_ref[...] + a1_ref[...] + b1_ref[...]
    z1 = _bn_relu(t, g1_ref[...], be1_ref[...])
    u = jnp.dot(z1, w2_ref[...], preferred_element_type=jnp.float32,
                 precision=lax.Precision.HIGHEST) \
        + b2_ref[...]
    z2 = _bn_relu(u, g2_ref[...], be2_ref[...])
    z2_ref[...] = z2
    yn_ref[...] = jnp.dot(z2, w1n_ref[...],
                          preferred_element_type=jnp.float32,
                 precision=lax.Precision.HIGHEST)


def _last_body(y_ref, a0_ref, a1_ref, b1_ref, g1_ref, be1_ref,
               w2_ref, b2_ref, g2_ref, be2_ref,
               x_ref, z0_ref, z1_ref, z2_ref,
               wox_ref, wo0_ref, wo1_ref, wo2_ref, wo3_ref, bo_ref,
               pred_ref):
    t = y_ref[...] + a0_ref[...] + a1_ref[...] + b1_ref[...]
    zz1 = _bn_relu(t, g1_ref[...], be1_ref[...])
    u = jnp.dot(zz1, w2_ref[...], preferred_element_type=jnp.float32) \
        + b2_ref[...]
    z3 = _bn_relu(u, g2_ref[...], be2_ref[...])
    pred = bo_ref[...]
    for lhs, w in ((x_ref[...], wox_ref[...]), (z0_ref[...], wo0_ref[...]),
                   (z1_ref[...], wo1_ref[...]), (z2_ref[...], wo2_ref[...]),
                   (z3, wo3_ref[...])):
        pred = pred + jnp.dot(lhs, w, preferred_element_type=jnp.float32)
    pred_ref[...] = pred


def _tc(body, out_shapes):
    return pl.pallas_call(body, out_shape=out_shapes)


def kernel(x, edge_index, params):
    src = edge_index[0]
    dst = edge_index[1]
    npad = _EPAD - N_EDGES
    src_p = jnp.concatenate([src, jnp.zeros((npad,), jnp.int32)])
    dst_p = jnp.concatenate([dst, jnp.full((npad,), N_NODES, jnp.int32)])
    zeros_init = jnp.zeros((_NPAD, HID), jnp.float32)

    layers = params["layers"]
    nd = lambda a: a.reshape(1, -1)

    y = _tc(_mm_body, jax.ShapeDtypeStruct((N_NODES, HID), jnp.float32))(
        x, layers[0]["W1"])

    zs = []
    for i in range(N_LAYERS - 1):
        lp = layers[i]
        agg2 = _sc_agg(y, src_p, dst_p, zeros_init)
        z2, y = _tc(_layer_body, (
            jax.ShapeDtypeStruct((N_NODES, HID), jnp.float32),
            jax.ShapeDtypeStruct((N_NODES, HID), jnp.float32),
        ))(y, agg2[0], agg2[1], nd(lp["b1"]), nd(lp["bn1_g"]),
           nd(lp["bn1_b"]), lp["W2"], nd(lp["b2"]), nd(lp["bn2_g"]),
           nd(lp["bn2_b"]), layers[i + 1]["W1"])
        zs.append(z2)

    lp = layers[N_LAYERS - 1]
    agg2 = _sc_agg(y, src_p, dst_p, zeros_init)
    w_out = params["W_out"]
    pred = _tc(_last_body, jax.ShapeDtypeStruct(
        (N_NODES, N_CLASSES), jnp.float32))(
        y, agg2[0], agg2[1], nd(lp["b1"]), nd(lp["bn1_g"]), nd(lp["bn1_b"]),
        lp["W2"], nd(lp["b2"]), nd(lp["bn2_g"]), nd(lp["bn2_b"]),
        x, zs[0], zs[1], zs[2],
        w_out[:IN_CH], w_out[IN_CH:IN_CH + HID],
        w_out[IN_CH + HID:IN_CH + 2 * HID],
        w_out[IN_CH + 2 * HID:IN_CH + 3 * HID],
        w_out[IN_CH + 3 * HID:],
        params["b_out"].reshape(1, -1))
    return pred
